# baseline (speedup 1.0000x reference)
import functools

import jax
import jax.numpy as jnp
from jax import lax
from jax.experimental import pallas as pl
from jax.experimental.pallas import tpu as pltpu

NUM_EXPERTS = 64
TILE = 512
LANES = 128
NEG_INF = float("-inf")


def _router_kernel(x_ref, w_ref, b_ref, rw_ref, se_ref):
    logits_t = lax.dot_general(
        w_ref[...], x_ref[0],
        dimension_numbers=(((1,), (1,)), ((), ())),
        preferred_element_type=jnp.float32,
    )
    logits_t = logits_t + b_ref[...]

    t = logits_t.shape[1]
    iota = lax.broadcasted_iota(jnp.int32, (NUM_EXPERTS, t), 0)
    big = jnp.int32(NUM_EXPERTS)

    m1 = jnp.max(logits_t, axis=0, keepdims=True)
    i1 = jnp.min(jnp.where(logits_t == m1, iota, big), axis=0, keepdims=True)
    masked = jnp.where(iota == i1, NEG_INF, logits_t)
    m2 = jnp.max(masked, axis=0, keepdims=True)
    i2 = jnp.min(jnp.where(masked == m2, iota, big), axis=0, keepdims=True)

    w1 = jax.nn.sigmoid(m1 - m2)
    w2 = 1.0 - w1

    nrow = t // LANES
    rw_ref[0, 0 : 2 * nrow : 2, :] = w1.reshape(nrow, LANES)
    rw_ref[0, 1 : 2 * nrow : 2, :] = w2.reshape(nrow, LANES)
    se_ref[0, 0 : 2 * nrow : 2, :] = i1.reshape(nrow, LANES)
    se_ref[0, 1 : 2 * nrow : 2, :] = i2.reshape(nrow, LANES)


@functools.partial(jax.jit, static_argnames=())
def _run(x, W, bcol):
    bsz, seq, d = x.shape
    nblk = seq // TILE
    rows_per_blk = 2 * TILE // LANES
    nrows = 2 * seq // LANES
    grid = (bsz, nblk)
    rw8, se8 = pl.pallas_call(
        _router_kernel,
        grid=grid,
        in_specs=[
            pl.BlockSpec((1, TILE, d), lambda bi, i: (bi, i, 0)),
            pl.BlockSpec((NUM_EXPERTS, d), lambda bi, i: (0, 0)),
            pl.BlockSpec((NUM_EXPERTS, 1), lambda bi, i: (0, 0)),
        ],
        out_specs=[
            pl.BlockSpec((1, rows_per_blk, LANES), lambda bi, i: (bi, i, 0)),
            pl.BlockSpec((1, rows_per_blk, LANES), lambda bi, i: (bi, i, 0)),
        ],
        out_shape=[
            jax.ShapeDtypeStruct((bsz, nrows, LANES), jnp.float32),
            jax.ShapeDtypeStruct((bsz, nrows, LANES), jnp.int32),
        ],
        compiler_params=pltpu.CompilerParams(
            dimension_semantics=("parallel", "parallel"),
        ),
    )(x, W, bcol)
    rw = rw8.reshape(bsz, seq // LANES, 2, LANES).swapaxes(2, 3).reshape(bsz, seq, 2)
    se = se8.reshape(bsz, seq // LANES, 2, LANES).swapaxes(2, 3).reshape(bsz, seq, 2)
    return rw, se


def kernel(x, W, b):
    bcol = b.reshape(NUM_EXPERTS, 1)
    return _run(x, W, bcol)


# speedup vs baseline: 1.0305x; 1.0305x over previous
import functools

import jax
import jax.numpy as jnp
from jax import lax
from jax.experimental import pallas as pl
from jax.experimental.pallas import tpu as pltpu

NUM_EXPERTS = 64
TILE = 1024
LANES = 128
NEG_INF = float("-inf")


def _router_kernel(x_ref, w_ref, b_ref, rw_ref, se_ref):
    logits_t = lax.dot_general(
        w_ref[...], x_ref[0],
        dimension_numbers=(((1,), (1,)), ((), ())),
        preferred_element_type=jnp.float32,
    )
    logits_t = logits_t + b_ref[...]

    t = logits_t.shape[1]
    iota = lax.broadcasted_iota(jnp.int32, (NUM_EXPERTS, t), 0)
    big = jnp.int32(NUM_EXPERTS)

    m1 = jnp.max(logits_t, axis=0, keepdims=True)
    i1 = jnp.min(jnp.where(logits_t == m1, iota, big), axis=0, keepdims=True)
    masked = jnp.where(iota == i1, NEG_INF, logits_t)
    m2 = jnp.max(masked, axis=0, keepdims=True)
    i2 = jnp.min(jnp.where(masked == m2, iota, big), axis=0, keepdims=True)

    w1 = jax.nn.sigmoid(m1 - m2)
    w2 = 1.0 - w1

    nrow = t // LANES
    rw_ref[0, 0 : 2 * nrow : 2, :] = w1.reshape(nrow, LANES)
    rw_ref[0, 1 : 2 * nrow : 2, :] = w2.reshape(nrow, LANES)
    se_ref[0, 0 : 2 * nrow : 2, :] = i1.reshape(nrow, LANES)
    se_ref[0, 1 : 2 * nrow : 2, :] = i2.reshape(nrow, LANES)


@functools.partial(jax.jit, static_argnames=())
def _run(x, W, bcol):
    bsz, seq, d = x.shape
    nblk = seq // TILE
    rows_per_blk = 2 * TILE // LANES
    nrows = 2 * seq // LANES
    grid = (bsz, nblk)
    rw8, se8 = pl.pallas_call(
        _router_kernel,
        grid=grid,
        in_specs=[
            pl.BlockSpec((1, TILE, d), lambda bi, i: (bi, i, 0)),
            pl.BlockSpec((NUM_EXPERTS, d), lambda bi, i: (0, 0)),
            pl.BlockSpec((NUM_EXPERTS, 1), lambda bi, i: (0, 0)),
        ],
        out_specs=[
            pl.BlockSpec((1, rows_per_blk, LANES), lambda bi, i: (bi, i, 0)),
            pl.BlockSpec((1, rows_per_blk, LANES), lambda bi, i: (bi, i, 0)),
        ],
        out_shape=[
            jax.ShapeDtypeStruct((bsz, nrows, LANES), jnp.float32),
            jax.ShapeDtypeStruct((bsz, nrows, LANES), jnp.int32),
        ],
        compiler_params=pltpu.CompilerParams(
            dimension_semantics=("parallel", "parallel"),
        ),
    )(x, W, bcol)
    rw = rw8.reshape(bsz, seq // LANES, 2, LANES).swapaxes(2, 3).reshape(bsz, seq, 2)
    se = se8.reshape(bsz, seq // LANES, 2, LANES).swapaxes(2, 3).reshape(bsz, seq, 2)
    return rw, se


def kernel(x, W, b):
    bcol = b.reshape(NUM_EXPERTS, 1)
    return _run(x, W, bcol)
